# Initial kernel scaffold; baseline (speedup 1.0000x reference)
import jax
import jax.numpy as jnp
from jax.experimental import pallas as pl


def kernel(x, w, b, gamma, beta):
    raise NotImplementedError("write your pallas kernel here")


# speedup vs baseline: 1.9483x; 1.9483x over previous
import functools

import jax
import jax.numpy as jnp
from jax.experimental import pallas as pl
from jax.experimental.pallas import tpu as pltpu

EPS = 1e-5


def _round_up(x, m):
    return (x + m - 1) // m * m


def _pick_row_tile(Ho, Wo):
    th = 1
    for cand in range(1, Ho + 1):
        if Ho % cand == 0 and cand * Wo <= 512:
            th = cand
    return th


def _make_fused_kernel(KH, KW, Ho, Wo, TH):
    TP = TH * Wo
    P = Ho * Wo
    n_chunks = Ho // TH
    n_full = P // 128
    rem = P - n_full * 128

    def _body(x_ref, w_ref, g_ref, bt_ref, o_ref, lhs_scr, y_scr):
        Cin = x_ref.shape[-1]
        CB = w_ref.shape[-1]
        w_mat = w_ref[...]

        s = jnp.zeros((1, CB), jnp.float32)
        ss = jnp.zeros((1, CB), jnp.float32)
        for c in range(n_chunks):
            h0 = c * TH
            for i in range(KH):
                for j in range(KW):
                    tap = x_ref[0, pl.ds(h0 + i, TH), pl.ds(j, Wo), :]
                    col = (i * KW + j) * Cin
                    lhs_scr[:, col:col + Cin] = tap.reshape(TP, Cin)
            y = jnp.dot(lhs_scr[...], w_mat,
                        preferred_element_type=jnp.float32)
            y_scr[c * TP:(c + 1) * TP, :] = y
            s = s + jnp.sum(y, axis=0, keepdims=True)
            ss = ss + jnp.sum(y * y, axis=0, keepdims=True)

        inv_p = 1.0 / float(P)
        mean = s * inv_p
        var = jnp.maximum(ss * inv_p - mean * mean, 0.0)
        scale = jax.lax.rsqrt(var + EPS) * g_ref[...]
        shift = bt_ref[...] - mean * scale

        for r in range(n_full):
            y = y_scr[r * 128:(r + 1) * 128, :]
            out = jnp.maximum(y * scale + shift, 0.0)
            o_ref[0, :, r * 128:(r + 1) * 128] = out.T
        if rem:
            y = y_scr[n_full * 128:P, :]
            out = jnp.maximum(y * scale + shift, 0.0)
            o_ref[0, :, n_full * 128:P] = out.T

    return _body


@functools.partial(jax.jit, static_argnames=("stride", "padding"))
def _fused_conv_in_relu(x_nchw, w_oihw, gamma, beta, *, stride=1, padding=0):
    N, Cin, H, W = x_nchw.shape
    Cout, Cin_w, KH, KW = w_oihw.shape
    assert Cin == Cin_w and stride == 1

    Ho = H + 2 * padding - KH + 1
    Wo = W + 2 * padding - KW + 1
    P = Ho * Wo
    K = KH * KW * Cin
    CB = 128
    Cp = _round_up(Cout, CB)
    n_cb = Cp // CB
    assert n_cb == 1, "single 128-channel block expected"

    x = jnp.transpose(x_nchw, (0, 2, 3, 1))
    if padding:
        x = jnp.pad(x, ((0, 0), (padding, padding), (padding, padding), (0, 0)))
    x = x.astype(jnp.bfloat16)
    Hp, Wp = x.shape[1], x.shape[2]

    w = jnp.transpose(w_oihw, (2, 3, 1, 0)).reshape(K, Cout)
    w = jnp.pad(w, ((0, 0), (0, Cp - Cout))).astype(jnp.bfloat16)
    gp = jnp.pad(gamma, (0, Cp - Cout)).reshape(1, Cp)
    btp = jnp.pad(beta, (0, Cp - Cout)).reshape(1, Cp)

    TH = _pick_row_tile(Ho, Wo)
    body = _make_fused_kernel(KH, KW, Ho, Wo, TH)

    out = pl.pallas_call(
        body,
        out_shape=jax.ShapeDtypeStruct((N, Cp, P), jnp.float32),
        grid=(N,),
        in_specs=[
            pl.BlockSpec((1, Hp, Wp, Cin), lambda n: (n, 0, 0, 0)),
            pl.BlockSpec((K, CB), lambda n: (0, 0)),
            pl.BlockSpec((1, CB), lambda n: (0, 0)),
            pl.BlockSpec((1, CB), lambda n: (0, 0)),
        ],
        out_specs=pl.BlockSpec((1, CB, P), lambda n: (n, 0, 0)),
        scratch_shapes=[
            pltpu.VMEM((TH * Wo, K), jnp.bfloat16),
            pltpu.VMEM((P, CB), jnp.float32),
        ],
        compiler_params=pltpu.CompilerParams(
            dimension_semantics=("parallel",)),
    )(x, w, gp, btp)

    return out[:, :Cout, :].reshape(N, Cout, Ho, Wo)


def kernel(x, w, b, gamma, beta):
    del b
    return _fused_conv_in_relu(x, w, gamma, beta, stride=1, padding=1)
